# baseline (speedup 1.0000x reference)
import functools

import numpy as np

import jax
import jax.numpy as jnp
from jax import lax
from jax.experimental import pallas as pl
from jax.experimental.pallas import tpu as pltpu
from jax.experimental.pallas import tpu_sc as plsc

N = 10000
B = 4
LEV = 2
NNZ = 320000
F = 128
FW = F // 2

NB = B - (LEV - 1)
NC = 2
NS = 16
NW = NC * NS
EPT = NNZ // NW
K = 128
BPT = 80
EPT_PAD = BPT * K
NBAT = NW * BPT
ROWS_PT = 624
EXTRA = N - NS * ROWS_PT
NBUF = 2
NSUB = 40
SUBS = BPT // NSUB

MASKHI = -65536

_PERM = np.empty(F, np.int32)
for _g in range(F // 32):
  for _k in range(16):
    _PERM[32 * _g + 2 * _k] = 32 * _g + _k
    _PERM[32 * _g + 2 * _k + 1] = 32 * _g + 16 + _k
_PERM.setflags(write=False)


def _pack_bf16(t):
  tb = t[..., _PERM].astype(jnp.bfloat16)
  return lax.bitcast_convert_type(
      tb.reshape(*tb.shape[:-1], FW, 2), jnp.int32)


def _make_spmm(per_block_out: bool):
  n_out = NB if per_block_out else 1
  mesh = plsc.VectorSubcoreMesh(core_axis_name="c", subcore_axis_name="s")
  scratch = [
      pltpu.VMEM_SHARED((N, F), jnp.float32),
      pltpu.VMEM((NSUB, K), jnp.int32),
      pltpu.VMEM((NSUB, K), jnp.int32),
      pltpu.VMEM((NSUB, K), jnp.float32),
      pltpu.VMEM((NBUF, K, FW), jnp.int32),
      pltpu.VMEM((K, F), jnp.float32),
  ] + [pltpu.SemaphoreType.DMA] * (NBUF + 1)

  @functools.partial(
      pl.kernel,
      out_type=jax.ShapeDtypeStruct((NC, n_out, N, F), jnp.float32),
      mesh=mesh,
      scratch_types=scratch,
      compiler_params=pltpu.CompilerParams(use_tc_tiling_on_sc=False),
  )
  def body(tbl, rows, cols, vals, zrows, out, acc, rbuf, cbuf, vbuf, gbuf,
           sbuf, *sems):
    gsems = sems[:NBUF]
    ssem = sems[NBUF]
    cid = lax.axis_index("c")
    sid = lax.axis_index("s")
    wid = cid * NS + sid
    row0 = sid * ROWS_PT

    def zero_acc():
      pltpu.sync_copy(zrows.at[pl.ds(0, ROWS_PT), :],
                      acc.at[pl.ds(row0, ROWS_PT), :])
      @pl.when(sid == NS - 1)
      def _():
        pltpu.sync_copy(zrows.at[pl.ds(0, EXTRA), :],
                        acc.at[pl.ds(NS * ROWS_PT, EXTRA), :])

    def writeout(ob):
      pltpu.sync_copy(acc.at[pl.ds(row0, ROWS_PT), :],
                      out.at[cid, ob, pl.ds(row0, ROWS_PT), :])
      @pl.when(sid == NS - 1)
      def _():
        sl = pl.ds(NS * ROWS_PT, EXTRA)
        pltpu.sync_copy(acc.at[sl, :], out.at[cid, ob, sl, :])

    def scale(u, s):
      def grp(g, carry):
        v16 = vbuf[s, pl.ds(g * 16, 16)]
        for e in range(16):
          be = jnp.take_along_axis(
              v16, jnp.full((16,), e, jnp.int32), 0,
              mode="promise_in_bounds")
          r = g * 16 + e
          for c in range(F // 32):
            w = gbuf[u, r, pl.ds(c * 16, 16)]
            lo = lax.bitcast_convert_type(
                lax.shift_left(w, jnp.full((16,), 16, jnp.int32)),
                jnp.float32)
            hi = lax.bitcast_convert_type(
                lax.bitwise_and(w, jnp.full((16,), MASKHI, jnp.int32)),
                jnp.float32)
            sbuf[r, pl.ds(c * 32, 16)] = lo * be
            sbuf[r, pl.ds(c * 32 + 16, 16)] = hi * be
        return carry
      lax.fori_loop(0, K // 16, grp, 0)

    def do_block(j):
      tj = 0 if per_block_out else j

      def gather_start(u, s):
        pltpu.async_copy(tbl.at[tj].at[cbuf.at[s]], gbuf.at[u], gsems[u])

      def gather_wait(u, s):
        pltpu.make_async_copy(
            tbl.at[tj].at[cbuf.at[s]], gbuf.at[u], gsems[u]).wait()

      def scatter_start(s):
        pltpu.async_copy(sbuf, acc.at[rbuf.at[s]], ssem, add=True)

      def scatter_wait(s):
        pltpu.make_async_copy(sbuf, acc.at[rbuf.at[s]], ssem).wait()

      def stage(st, carry):
        base = wid * BPT + st * NSUB
        pltpu.sync_copy(
            (rows.at[j, pl.ds(base, NSUB), :],
             cols.at[j, pl.ds(base, NSUB), :],
             vals.at[j, pl.ds(base, NSUB), :]),
            (rbuf, cbuf, vbuf),
        )
        for u in range(NBUF):
          gather_start(u, u)

        def ring(i, c2):
          for u in range(NBUF):
            s = i * NBUF + u
            gather_wait(u, s)
            @pl.when(s > 0)
            def _():
              scatter_wait(s - 1)
            scale(u, s)
            @pl.when(s + NBUF < NSUB)
            def _():
              gather_start(u, s + NBUF)
            scatter_start(s)
          return c2
        lax.fori_loop(0, NSUB // NBUF, ring, 0)

        scatter_wait(NSUB - 1)
        return carry
      lax.fori_loop(0, SUBS, stage, 0)

    if per_block_out:
      def blk(j, carry):
        zero_acc()
        plsc.subcore_barrier()
        do_block(j)
        plsc.subcore_barrier()
        writeout(j)
        return carry
      lax.fori_loop(0, NB, blk, 0)
    else:
      zero_acc()
      plsc.subcore_barrier()
      def blk(j, carry):
        do_block(j)
        return carry
      lax.fori_loop(0, NB, blk, 0)
      plsc.subcore_barrier()
      writeout(0)

  return body


def _matmul(x, w):
  def mm(x_ref, w_ref, o_ref):
    o_ref[...] = jnp.dot(x_ref[...], w_ref[...],
                         preferred_element_type=jnp.float32)
  return pl.pallas_call(
      mm,
      grid=(10,),
      in_specs=[pl.BlockSpec((N // 10, F), lambda i: (i, 0)),
                pl.BlockSpec((F, F), lambda i: (0, 0))],
      out_specs=pl.BlockSpec((N // 10, F), lambda i: (i, 0)),
      out_shape=jax.ShapeDtypeStruct((N, F), jnp.float32),
  )(x, w)


def _merge(yp, filt3):
  def m(a_ref, b_ref, f_ref, o_ref):
    o_ref[...] = f_ref[...] * (a_ref[...] + b_ref[...])
  return pl.pallas_call(
      m,
      grid=(NB, 10),
      in_specs=[
          pl.BlockSpec((1, N // 10, F), lambda j, i: (j, i, 0)),
          pl.BlockSpec((1, N // 10, F), lambda j, i: (j, i, 0)),
          pl.BlockSpec((1, N // 10, 1), lambda j, i: (j, i, 0)),
      ],
      out_specs=pl.BlockSpec((1, N // 10, F), lambda j, i: (j, i, 0)),
      out_shape=jax.ShapeDtypeStruct((NB, N, F), jnp.float32),
  )(yp[0], yp[1], filt3)


def _final(z0, z1, bias2):
  def f(a_ref, b_ref, bias_ref, o_ref):
    o_ref[...] = a_ref[...] + b_ref[...] + bias_ref[...]
  return pl.pallas_call(
      f,
      grid=(10,),
      in_specs=[pl.BlockSpec((N // 10, F), lambda i: (i, 0)),
                pl.BlockSpec((N // 10, F), lambda i: (i, 0)),
                pl.BlockSpec((1, F), lambda i: (0, 0))],
      out_specs=pl.BlockSpec((N // 10, F), lambda i: (i, 0)),
      out_shape=jax.ShapeDtypeStruct((N, F), jnp.float32),
  )(z0, z1, bias2)


def _prep(d_row, d_col, d_vals):
  pade = EPT_PAD - EPT
  spread = (jnp.arange(NW * pade, dtype=jnp.int32) * 37 + 11) % N
  pad_idx = jnp.broadcast_to(spread.reshape(1, NW, pade), (NB, NW, pade))

  def pad3(a, pad):
    a3 = a[LEV - 1:].reshape(NB, NW, EPT)
    return jnp.concatenate([a3, pad], axis=2).reshape(NB, NBAT, K)

  rows = pad3(d_row, pad_idx)
  cols = pad3(d_col, pad_idx)
  vals = pad3(d_vals, jnp.zeros((NB, NW, pade), jnp.float32))
  return rows, cols, vals


def kernel(x, d_row, d_col, d_vals, weight, filt, bias):
  h = _matmul(x, weight)
  rows, cols, vals = _prep(d_row, d_col, d_vals)
  zrows = jnp.zeros((ROWS_PT, F), jnp.float32)
  tbl_a = _pack_bf16(h)[None]
  yp = _make_spmm(True)(tbl_a, rows, cols, vals, zrows)
  filt3 = filt[(LEV - 1) * N:].reshape(NB, N, 1)
  yf = _merge(yp, filt3)
  tbl_b = _pack_bf16(yf)
  zp = _make_spmm(False)(tbl_b, rows, cols, vals, zrows)
  return _final(zp[0, 0], zp[1, 0], bias.reshape(1, F))


# speedup vs baseline: 2.2328x; 2.2328x over previous
import functools

import jax
import jax.numpy as jnp
from jax import lax
from jax.experimental import pallas as pl
from jax.experimental.pallas import tpu as pltpu
from jax.experimental.pallas import tpu_sc as plsc

N = 10000
B = 4
LEV = 2
NNZ = 320000
F = 128

NB = B - (LEV - 1)
NC = 2
NS = 16
NW = NC * NS
EPT = NNZ // NW
K = 128
BPT = 80
EPT_PAD = BPT * K
NBAT = NW * BPT
ROWS_PT = 624
EXTRA = N - NS * ROWS_PT
NBUF = 2
NSUB = 40
SUBS = BPT // NSUB


def _make_spmm(per_block_out: bool):
  n_out = NB if per_block_out else 1
  mesh = plsc.VectorSubcoreMesh(core_axis_name="c", subcore_axis_name="s")
  scratch = [
      pltpu.VMEM_SHARED((N, F), jnp.float32),
      pltpu.VMEM((NSUB, K), jnp.int32),
      pltpu.VMEM((NSUB, K), jnp.int32),
      pltpu.VMEM((NSUB, K), jnp.float32),
      pltpu.VMEM((NBUF, K, F), jnp.float32),
  ] + [pltpu.SemaphoreType.DMA] * (2 * NBUF)

  @functools.partial(
      pl.kernel,
      out_type=jax.ShapeDtypeStruct((NC, n_out, N, F), jnp.float32),
      mesh=mesh,
      scratch_types=scratch,
  )
  def body(tbl, rows, cols, vals, zrows, out, acc, rbuf, cbuf, vbuf, gbuf,
           *sems):
    gsems = sems[:NBUF]
    ssems = sems[NBUF:]
    cid = lax.axis_index("c")
    sid = lax.axis_index("s")
    wid = cid * NS + sid
    row0 = sid * ROWS_PT

    def zero_acc():
      pltpu.sync_copy(zrows.at[pl.ds(0, ROWS_PT), :],
                      acc.at[pl.ds(row0, ROWS_PT), :])
      @pl.when(sid == NS - 1)
      def _():
        pltpu.sync_copy(zrows.at[pl.ds(0, EXTRA), :],
                        acc.at[pl.ds(NS * ROWS_PT, EXTRA), :])

    def writeout(ob):
      pltpu.sync_copy(acc.at[pl.ds(row0, ROWS_PT), :],
                      out.at[cid, ob, pl.ds(row0, ROWS_PT), :])
      @pl.when(sid == NS - 1)
      def _():
        sl = pl.ds(NS * ROWS_PT, EXTRA)
        pltpu.sync_copy(acc.at[sl, :], out.at[cid, ob, sl, :])

    def scale(u, s):
      def grp(g, carry):
        v16 = vbuf[s, pl.ds(g * 16, 16)]
        for e in range(16):
          be = jnp.take_along_axis(
              v16, jnp.full((16,), e, jnp.int32), 0,
              mode="promise_in_bounds")
          r = g * 16 + e
          for c in range(F // 16):
            sl = pl.ds(c * 16, 16)
            gbuf[u, r, sl] = gbuf[u, r, sl] * be
        return carry
      lax.fori_loop(0, K // 16, grp, 0)

    def do_block(j):
      tj = 0 if per_block_out else j

      def gather_start(u, s):
        pltpu.async_copy(tbl.at[tj].at[cbuf.at[s]], gbuf.at[u], gsems[u])

      def gather_wait(u, s):
        pltpu.make_async_copy(
            tbl.at[tj].at[cbuf.at[s]], gbuf.at[u], gsems[u]).wait()

      def scatter_start(u, s):
        pltpu.async_copy(gbuf.at[u], acc.at[rbuf.at[s]], ssems[u], add=True)

      def scatter_wait(u, s):
        pltpu.make_async_copy(gbuf.at[u], acc.at[rbuf.at[s]], ssems[u]).wait()

      def stage(st, carry):
        base = wid * BPT + st * NSUB
        pltpu.sync_copy(
            (rows.at[j, pl.ds(base, NSUB), :],
             cols.at[j, pl.ds(base, NSUB), :],
             vals.at[j, pl.ds(base, NSUB), :]),
            (rbuf, cbuf, vbuf),
        )
        for u in range(NBUF):
          gather_start(u, u)

        def ring(i, c2):
          for u in range(NBUF):
            s = i * NBUF + u
            gather_wait(u, s)
            up = (u - 1) % NBUF
            sp = s - 1
            @pl.when(jnp.logical_and(sp >= 0, sp + NBUF < NSUB))
            def _():
              scatter_wait(up, sp)
              gather_start(up, sp + NBUF)
            scale(u, s)
            scatter_start(u, s)
          return c2
        lax.fori_loop(0, NSUB // NBUF, ring, 0)

        for u in range(NBUF):
          scatter_wait(u, NSUB - NBUF + u)
        return carry
      lax.fori_loop(0, SUBS, stage, 0)

    if per_block_out:
      def blk(j, carry):
        zero_acc()
        plsc.subcore_barrier()
        do_block(j)
        plsc.subcore_barrier()
        writeout(j)
        return carry
      lax.fori_loop(0, NB, blk, 0)
    else:
      zero_acc()
      plsc.subcore_barrier()
      def blk(j, carry):
        do_block(j)
        return carry
      lax.fori_loop(0, NB, blk, 0)
      plsc.subcore_barrier()
      writeout(0)

  return body


def _matmul(x, w):
  def mm(x_ref, w_ref, o_ref):
    o_ref[...] = jnp.dot(x_ref[...], w_ref[...],
                         preferred_element_type=jnp.float32)
  return pl.pallas_call(
      mm,
      grid=(10,),
      in_specs=[pl.BlockSpec((N // 10, F), lambda i: (i, 0)),
                pl.BlockSpec((F, F), lambda i: (0, 0))],
      out_specs=pl.BlockSpec((N // 10, F), lambda i: (i, 0)),
      out_shape=jax.ShapeDtypeStruct((N, F), jnp.float32),
  )(x, w)


def _merge(yp, filt3):
  def m(a_ref, b_ref, f_ref, o_ref):
    o_ref[...] = f_ref[...] * (a_ref[...] + b_ref[...])
  return pl.pallas_call(
      m,
      grid=(NB, 10),
      in_specs=[
          pl.BlockSpec((1, N // 10, F), lambda j, i: (j, i, 0)),
          pl.BlockSpec((1, N // 10, F), lambda j, i: (j, i, 0)),
          pl.BlockSpec((1, N // 10, 1), lambda j, i: (j, i, 0)),
      ],
      out_specs=pl.BlockSpec((1, N // 10, F), lambda j, i: (j, i, 0)),
      out_shape=jax.ShapeDtypeStruct((NB, N, F), jnp.float32),
  )(yp[0], yp[1], filt3)


def _final(z0, z1, bias2):
  def f(a_ref, b_ref, bias_ref, o_ref):
    o_ref[...] = a_ref[...] + b_ref[...] + bias_ref[...]
  return pl.pallas_call(
      f,
      grid=(10,),
      in_specs=[pl.BlockSpec((N // 10, F), lambda i: (i, 0)),
                pl.BlockSpec((N // 10, F), lambda i: (i, 0)),
                pl.BlockSpec((1, F), lambda i: (0, 0))],
      out_specs=pl.BlockSpec((N // 10, F), lambda i: (i, 0)),
      out_shape=jax.ShapeDtypeStruct((N, F), jnp.float32),
  )(z0, z1, bias2)


def _prep(d_row, d_col, d_vals):
  pade = EPT_PAD - EPT
  spread = (jnp.arange(NW * pade, dtype=jnp.int32) * 37 + 11) % N
  pad_idx = jnp.broadcast_to(spread.reshape(1, NW, pade), (NB, NW, pade))

  def pad3(a, pad):
    a3 = a[LEV - 1:].reshape(NB, NW, EPT)
    return jnp.concatenate([a3, pad], axis=2).reshape(NB, NBAT, K)

  rows = pad3(d_row, pad_idx)
  cols = pad3(d_col, pad_idx)
  vals = pad3(d_vals, jnp.zeros((NB, NW, pade), jnp.float32))
  return rows, cols, vals


def kernel(x, d_row, d_col, d_vals, weight, filt, bias):
  h = _matmul(x, weight)
  rows, cols, vals = _prep(d_row, d_col, d_vals)
  zrows = jnp.zeros((ROWS_PT, F), jnp.float32)
  yp = _make_spmm(True)(h[None], rows, cols, vals, zrows)
  filt3 = filt[(LEV - 1) * N:].reshape(NB, N, 1)
  yf = _merge(yp, filt3)
  zp = _make_spmm(False)(yf, rows, cols, vals, zrows)
  return _final(zp[0, 0], zp[1, 0], bias.reshape(1, F))
